# baseline (speedup 1.0000x reference)
import functools

import jax
import jax.numpy as jnp
from jax import lax
from jax.experimental import pallas as pl
from jax.experimental.pallas import tpu as pltpu
from jax.experimental.pallas import tpu_sc as plsc

N_NODES = 10000
N_EDGES = 320000
D_FEAT = 128
HID = 64
N_GRAPHS = 64
N_LAYERS = 3

NC = 2
NS = 16
NW = NC * NS
CH = 128
NCH = 80
EPW = NCH * CH
N_EDGES_PAD = NW * EPW
NBUF = 2
N_SCRAP = 128
ACC_ROWS = N_NODES + N_SCRAP
RPS = 624
TAIL0 = NS * RPS
TAILZ = ACC_ROWS - TAIL0
TAILO = N_NODES - TAIL0

_f32 = jnp.float32


def _segsum_body(h_hbm, src_hbm, dst_hbm, zero_hbm, out_hbm,
                 src_v, dst_v, rows_v, acc, h_sp, gsems):
    c = lax.axis_index("c")
    s = lax.axis_index("s")
    wid = s * NC + c

    pltpu.sync_copy(src_hbm.at[wid], src_v)
    pltpu.sync_copy(dst_hbm.at[wid], dst_v)

    r0 = s * RPS
    pltpu.sync_copy(zero_hbm.at[pl.ds(r0, RPS)], acc.at[pl.ds(r0, RPS)])
    pltpu.sync_copy(h_hbm.at[pl.ds(r0, RPS)], h_sp.at[pl.ds(r0, RPS)])

    @pl.when(s == NS - 1)
    def _():
        pltpu.sync_copy(zero_hbm.at[pl.ds(TAIL0, TAILZ)],
                        acc.at[pl.ds(TAIL0, TAILZ)])
        pltpu.sync_copy(h_hbm.at[pl.ds(TAIL0, TAILO)],
                        h_sp.at[pl.ds(TAIL0, TAILO)])

    plsc.subcore_barrier()

    for b in range(NBUF):
        pltpu.async_copy(h_sp.at[src_v.at[b]], rows_v.at[b], gsems.at[b])

    n_outer = NCH // NBUF

    def outer(g, carry):
        for b in range(NBUF):
            j = g * NBUF + b
            pltpu.make_async_copy(
                h_sp.at[src_v.at[j]], rows_v.at[b], gsems.at[b]).wait()
            pltpu.sync_copy(rows_v.at[b], acc.at[dst_v.at[j]], add=True)

            @pl.when(g < n_outer - 1)
            def _():
                pltpu.async_copy(
                    h_sp.at[src_v.at[j + NBUF]], rows_v.at[b], gsems.at[b])
        return carry

    lax.fori_loop(0, n_outer, outer, 0)
    plsc.subcore_barrier()

    pltpu.sync_copy(acc.at[pl.ds(r0, RPS)],
                    out_hbm.at[pl.ds(r0, RPS), pl.ds(c * HID, HID)])

    @pl.when(s == NS - 1)
    def _():
        pltpu.sync_copy(acc.at[pl.ds(TAIL0, TAILO)],
                        out_hbm.at[pl.ds(TAIL0, TAILO), pl.ds(c * HID, HID)])


@functools.cache
def _get_segsum():
    return pl.kernel(
        _segsum_body,
        out_type=jax.ShapeDtypeStruct((N_NODES, 2 * HID), _f32),
        mesh=plsc.VectorSubcoreMesh(core_axis_name="c", subcore_axis_name="s"),
        scratch_types=[
            pltpu.VMEM((NCH, CH), jnp.int32),
            pltpu.VMEM((NCH, CH), jnp.int32),
            pltpu.VMEM((NBUF, CH, HID), _f32),
            pltpu.VMEM_SHARED((ACC_ROWS, HID), _f32),
            pltpu.VMEM_SHARED((N_NODES, HID), _f32),
            pltpu.SemaphoreType.DMA((NBUF,)),
        ],
        compiler_params=pltpu.CompilerParams(use_tc_tiling_on_sc=False),
    )


def _bn_relu(t, g, be):
    m = jnp.mean(t, axis=0, keepdims=True)
    v = jnp.mean(t * t, axis=0, keepdims=True) - m * m
    return jnp.maximum(g * (t - m) / jnp.sqrt(v + 1e-5) + be, 0.0)


def _mlp(y, w1, b1, g1, be1, w2, b2, g2, be2):
    t = jnp.dot(y, w1, preferred_element_type=_f32) + b1
    t = _bn_relu(t, g1, be1)
    u = jnp.dot(t, w2, preferred_element_type=_f32) + b2
    return _bn_relu(u, g2, be2)


def _onehot(batch):
    gids = lax.broadcasted_iota(jnp.int32, (N_NODES, N_GRAPHS), 1)
    return (batch == gids).astype(_f32)


def _pool_sum(h, wl, onehot):
    z = jnp.sum(h * wl, axis=1, keepdims=True)
    return lax.dot_general(z, onehot, (((0,), (0,)), ((), ())),
                           preferred_element_type=_f32)


def _first_body(x_ref, w1, b1, g1, be1, w2, b2, g2, be2, h_ref):
    h_ref[...] = _mlp(x_ref[...], w1[...], b1[...], g1[...], be1[...],
                      w2[...], b2[...], g2[...], be2[...])


_first = pl.pallas_call(
    _first_body,
    out_shape=jax.ShapeDtypeStruct((N_NODES, HID), _f32),
)


def _pool0_body(h_ref, wl, batch_ref, sc_ref):
    onehot = _onehot(batch_ref[...])
    z = jnp.sum(h_ref[...] * wl[...], axis=1, keepdims=True)
    zo = jnp.concatenate([z, jnp.ones_like(z)], axis=1)
    sc_ref[...] = lax.dot_general(zo, onehot, (((0,), (0,)), ((), ())),
                                  preferred_element_type=_f32)


_pool0 = pl.pallas_call(
    _pool0_body,
    out_shape=jax.ShapeDtypeStruct((2, N_GRAPHS), _f32),
)


def _pooln_body(h_ref, wl, batch_ref, s_ref):
    s_ref[...] = _pool_sum(h_ref[...], wl[...], _onehot(batch_ref[...]))


_pooln = pl.pallas_call(
    _pooln_body,
    out_shape=jax.ShapeDtypeStruct((1, N_GRAPHS), _f32),
)


def _mid_body(h_ref, p_ref, w1, b1, g1, be1, w2, b2, g2, be2, h_ref_out):
    p = p_ref[...]
    y = h_ref[...] + p[:, :HID] + p[:, HID:]
    h_ref_out[...] = _mlp(y, w1[...], b1[...], g1[...], be1[...],
                          w2[...], b2[...], g2[...], be2[...])


_mid = pl.pallas_call(
    _mid_body,
    out_shape=jax.ShapeDtypeStruct((N_NODES, HID), _f32),
)


def _last_body(h_ref, p_ref, w1, b1, g1, be1, w2, b2, g2, be2, wl, batch_ref,
               sc_ref, s1_ref, s2_ref, bsum_ref, out_ref):
    p = p_ref[...]
    y = h_ref[...] + p[:, :HID] + p[:, HID:]
    h = _mlp(y, w1[...], b1[...], g1[...], be1[...],
             w2[...], b2[...], g2[...], be2[...])
    s3 = _pool_sum(h, wl[...], _onehot(batch_ref[...]))
    s0 = sc_ref[0:1, :]
    cnt = sc_ref[1:2, :]
    total = s0 + s1_ref[...] + s2_ref[...] + s3 + bsum_ref[0, 0] * cnt
    out_ref[...] = jax.nn.sigmoid(total / jnp.maximum(cnt, 1.0))


_last = pl.pallas_call(
    _last_body,
    out_shape=jax.ShapeDtypeStruct((1, N_GRAPHS), _f32),
)


def kernel(x, edge_index, batch, params):
    npad = N_EDGES_PAD - N_EDGES
    spread = jnp.arange(npad, dtype=jnp.int32) % N_SCRAP
    srcp = jnp.concatenate(
        [edge_index[0], spread]).reshape(NW, NCH, CH)
    dstp = jnp.concatenate(
        [edge_index[1], N_NODES + spread]).reshape(NW, NCH, CH)
    zeros = jnp.zeros((ACC_ROWS, HID), _f32)
    batch2 = batch.reshape(N_NODES, 1)
    bsum = (params['lin_b'][0] + params['lin_b'][1] + params['lin_b'][2]
            + params['lin_b'][3]).reshape(1, 1)

    def row(v):
        return v.reshape(1, HID)

    def mlp_args(p):
        return (p['W1'], row(p['b1']), row(p['g1']), row(p['be1']),
                p['W2'], row(p['b2']), row(p['g2']), row(p['be2']))

    segsum = _get_segsum()
    h = _first(x, *mlp_args(params['first_h']))
    parts = segsum(h, srcp, dstp, zeros)
    sc = _pool0(h, params['lin_W'][0].reshape(1, HID), batch2)
    h = _mid(h, parts, *mlp_args(params['nns'][0]))
    parts = segsum(h, srcp, dstp, zeros)
    s1 = _pooln(h, params['lin_W'][1].reshape(1, HID), batch2)
    h = _mid(h, parts, *mlp_args(params['nns'][1]))
    parts = segsum(h, srcp, dstp, zeros)
    s2 = _pooln(h, params['lin_W'][2].reshape(1, HID), batch2)
    out = _last(h, parts, *mlp_args(params['nns'][2]),
                params['lin_W'][3].reshape(1, HID), batch2,
                sc, s1, s2, bsum)
    return out.reshape(-1)


# speedup vs baseline: 1.3753x; 1.3753x over previous
import functools

import jax
import jax.numpy as jnp
from jax import lax
from jax.experimental import pallas as pl
from jax.experimental.pallas import tpu as pltpu
from jax.experimental.pallas import tpu_sc as plsc

N_NODES = 10000
N_EDGES = 320000
D_FEAT = 128
HID = 64
N_GRAPHS = 64
N_LAYERS = 3

NC = 2
NS = 16
NW = NC * NS
CH = 128
NCH = 80
EPW = NCH * CH
N_EDGES_PAD = NW * EPW
NBUF = 5
N_SCRAP = 128
ACC_ROWS = N_NODES + N_SCRAP
RPS = 624
TAIL0 = NS * RPS
TAILZ = ACC_ROWS - TAIL0
TAILO = N_NODES - TAIL0

_f32 = jnp.float32
_bf16 = jnp.bfloat16


def _segsum_body(h_hbm, src_hbm, dst_hbm, zero_hbm, out_hbm,
                 src_v, dst_v, rows_v, acc, gsems):
    c = lax.axis_index("c")
    s = lax.axis_index("s")
    wid = s * NC + c

    pltpu.sync_copy(src_hbm.at[wid], src_v)
    pltpu.sync_copy(dst_hbm.at[wid], dst_v)

    r0 = s * RPS
    pltpu.sync_copy(zero_hbm.at[pl.ds(r0, RPS)], acc.at[pl.ds(r0, RPS)])

    @pl.when(s == NS - 1)
    def _():
        pltpu.sync_copy(zero_hbm.at[pl.ds(TAIL0, TAILZ)],
                        acc.at[pl.ds(TAIL0, TAILZ)])

    plsc.subcore_barrier()

    for b in range(NBUF):
        pltpu.async_copy(h_hbm.at[src_v.at[b]], rows_v.at[b], gsems.at[b])

    n_outer = NCH // NBUF

    def outer(g, carry):
        for b in range(NBUF):
            j = g * NBUF + b
            pltpu.make_async_copy(
                h_hbm.at[src_v.at[j]], rows_v.at[b], gsems.at[b]).wait()
            pltpu.sync_copy(rows_v.at[b], acc.at[dst_v.at[j]], add=True)

            @pl.when(g < n_outer - 1)
            def _():
                pltpu.async_copy(
                    h_hbm.at[src_v.at[j + NBUF]], rows_v.at[b], gsems.at[b])
        return carry

    lax.fori_loop(0, n_outer, outer, 0)
    plsc.subcore_barrier()

    pltpu.sync_copy(acc.at[pl.ds(r0, RPS)],
                    out_hbm.at[pl.ds(r0, RPS), pl.ds(c * HID, HID)])

    @pl.when(s == NS - 1)
    def _():
        pltpu.sync_copy(acc.at[pl.ds(TAIL0, TAILO)],
                        out_hbm.at[pl.ds(TAIL0, TAILO), pl.ds(c * HID, HID)])


@functools.cache
def _get_segsum():
    return pl.kernel(
        _segsum_body,
        out_type=jax.ShapeDtypeStruct((N_NODES, 2 * HID), _bf16),
        mesh=plsc.VectorSubcoreMesh(core_axis_name="c", subcore_axis_name="s"),
        scratch_types=[
            pltpu.VMEM((NCH, CH), jnp.int32),
            pltpu.VMEM((NCH, CH), jnp.int32),
            pltpu.VMEM((NBUF, CH, HID), _bf16),
            pltpu.VMEM_SHARED((ACC_ROWS, HID), _bf16),
            pltpu.SemaphoreType.DMA((NBUF,)),
        ],
        compiler_params=pltpu.CompilerParams(use_tc_tiling_on_sc=False),
    )


def _bn_relu(t, g, be):
    m = jnp.mean(t, axis=0, keepdims=True)
    v = jnp.mean(t * t, axis=0, keepdims=True) - m * m
    return jnp.maximum(g * (t - m) / jnp.sqrt(v + 1e-5) + be, 0.0)


def _mlp(y, w1, b1, g1, be1, w2, b2, g2, be2):
    t = jnp.dot(y, w1, preferred_element_type=_f32) + b1
    t = _bn_relu(t, g1, be1)
    u = jnp.dot(t, w2, preferred_element_type=_f32) + b2
    return _bn_relu(u, g2, be2)


def _onehot(batch):
    gids = lax.broadcasted_iota(jnp.int32, (N_NODES, N_GRAPHS), 1)
    return (batch == gids).astype(_f32)


def _pool_sum(h, wl, onehot):
    z = jnp.sum(h * wl, axis=1, keepdims=True)
    return lax.dot_general(z, onehot, (((0,), (0,)), ((), ())),
                           preferred_element_type=_f32)


def _first_body(x_ref, w1, b1, g1, be1, w2, b2, g2, be2, h_ref, hb_ref):
    h = _mlp(x_ref[...], w1[...], b1[...], g1[...], be1[...],
             w2[...], b2[...], g2[...], be2[...])
    h_ref[...] = h
    hb_ref[...] = h.astype(_bf16)


_first = pl.pallas_call(
    _first_body,
    out_shape=(jax.ShapeDtypeStruct((N_NODES, HID), _f32),
               jax.ShapeDtypeStruct((N_NODES, HID), _bf16)),
)


def _pool0_body(h_ref, wl, batch_ref, sc_ref):
    onehot = _onehot(batch_ref[...])
    z = jnp.sum(h_ref[...] * wl[...], axis=1, keepdims=True)
    zo = jnp.concatenate([z, jnp.ones_like(z)], axis=1)
    sc_ref[...] = lax.dot_general(zo, onehot, (((0,), (0,)), ((), ())),
                                  preferred_element_type=_f32)


_pool0 = pl.pallas_call(
    _pool0_body,
    out_shape=jax.ShapeDtypeStruct((2, N_GRAPHS), _f32),
)


def _pooln_body(h_ref, wl, batch_ref, s_ref):
    s_ref[...] = _pool_sum(h_ref[...], wl[...], _onehot(batch_ref[...]))


_pooln = pl.pallas_call(
    _pooln_body,
    out_shape=jax.ShapeDtypeStruct((1, N_GRAPHS), _f32),
)


def _mid_body(h_ref, p_ref, w1, b1, g1, be1, w2, b2, g2, be2,
              h_ref_out, hb_ref):
    p = p_ref[...].astype(_f32)
    y = h_ref[...] + p[:, :HID] + p[:, HID:]
    h = _mlp(y, w1[...], b1[...], g1[...], be1[...],
             w2[...], b2[...], g2[...], be2[...])
    h_ref_out[...] = h
    hb_ref[...] = h.astype(_bf16)


_mid = pl.pallas_call(
    _mid_body,
    out_shape=(jax.ShapeDtypeStruct((N_NODES, HID), _f32),
               jax.ShapeDtypeStruct((N_NODES, HID), _bf16)),
)


def _last_body(h_ref, p_ref, w1, b1, g1, be1, w2, b2, g2, be2, wl, batch_ref,
               sc_ref, s1_ref, s2_ref, bsum_ref, out_ref):
    p = p_ref[...].astype(_f32)
    y = h_ref[...] + p[:, :HID] + p[:, HID:]
    h = _mlp(y, w1[...], b1[...], g1[...], be1[...],
             w2[...], b2[...], g2[...], be2[...])
    s3 = _pool_sum(h, wl[...], _onehot(batch_ref[...]))
    s0 = sc_ref[0:1, :]
    cnt = sc_ref[1:2, :]
    total = s0 + s1_ref[...] + s2_ref[...] + s3 + bsum_ref[0, 0] * cnt
    out_ref[...] = jax.nn.sigmoid(total / jnp.maximum(cnt, 1.0))


_last = pl.pallas_call(
    _last_body,
    out_shape=jax.ShapeDtypeStruct((1, N_GRAPHS), _f32),
)


def kernel(x, edge_index, batch, params):
    npad = N_EDGES_PAD - N_EDGES
    spread = jnp.arange(npad, dtype=jnp.int32) % N_SCRAP
    srcp = jnp.concatenate(
        [edge_index[0], spread]).reshape(NW, NCH, CH)
    dstp = jnp.concatenate(
        [edge_index[1], N_NODES + spread]).reshape(NW, NCH, CH)
    zeros = jnp.zeros((ACC_ROWS, HID), _bf16)
    batch2 = batch.reshape(N_NODES, 1)
    bsum = (params['lin_b'][0] + params['lin_b'][1] + params['lin_b'][2]
            + params['lin_b'][3]).reshape(1, 1)

    def row(v):
        return v.reshape(1, HID)

    def mlp_args(p):
        return (p['W1'], row(p['b1']), row(p['g1']), row(p['be1']),
                p['W2'], row(p['b2']), row(p['g2']), row(p['be2']))

    segsum = _get_segsum()
    h, hb = _first(x, *mlp_args(params['first_h']))
    parts = segsum(hb, srcp, dstp, zeros)
    sc = _pool0(h, params['lin_W'][0].reshape(1, HID), batch2)
    h, hb = _mid(h, parts, *mlp_args(params['nns'][0]))
    parts = segsum(hb, srcp, dstp, zeros)
    s1 = _pooln(h, params['lin_W'][1].reshape(1, HID), batch2)
    h, hb = _mid(h, parts, *mlp_args(params['nns'][1]))
    parts = segsum(hb, srcp, dstp, zeros)
    s2 = _pooln(h, params['lin_W'][2].reshape(1, HID), batch2)
    out = _last(h, parts, *mlp_args(params['nns'][2]),
                params['lin_W'][3].reshape(1, HID), batch2,
                sc, s1, s2, bsum)
    return out.reshape(-1)


# speedup vs baseline: 1.3778x; 1.0018x over previous
import functools

import jax
import jax.numpy as jnp
from jax import lax
from jax.experimental import pallas as pl
from jax.experimental.pallas import tpu as pltpu
from jax.experimental.pallas import tpu_sc as plsc

N_NODES = 10000
N_EDGES = 320000
D_FEAT = 128
HID = 64
N_GRAPHS = 64
N_LAYERS = 3

NC = 2
NS = 16
NW = NC * NS
CH = 128
NCH = 80
EPW = NCH * CH
N_EDGES_PAD = NW * EPW
NBUF = 10
N_SCRAP = 128
ACC_ROWS = N_NODES + N_SCRAP
RPS = 624
TAIL0 = NS * RPS
TAILZ = ACC_ROWS - TAIL0
TAILO = N_NODES - TAIL0

_f32 = jnp.float32
_bf16 = jnp.bfloat16


def _segsum_body(h_hbm, src_hbm, dst_hbm, zero_hbm, out_hbm,
                 src_v, dst_v, rows_v, acc, gsems):
    c = lax.axis_index("c")
    s = lax.axis_index("s")
    wid = s * NC + c

    pltpu.sync_copy(src_hbm.at[wid], src_v)
    pltpu.sync_copy(dst_hbm.at[wid], dst_v)

    r0 = s * RPS
    pltpu.sync_copy(zero_hbm.at[pl.ds(r0, RPS)], acc.at[pl.ds(r0, RPS)])

    @pl.when(s == NS - 1)
    def _():
        pltpu.sync_copy(zero_hbm.at[pl.ds(TAIL0, TAILZ)],
                        acc.at[pl.ds(TAIL0, TAILZ)])

    plsc.subcore_barrier()

    for b in range(NBUF):
        pltpu.async_copy(h_hbm.at[src_v.at[b]], rows_v.at[b], gsems.at[b])

    n_outer = NCH // NBUF

    def outer(g, carry):
        for b in range(NBUF):
            j = g * NBUF + b
            pltpu.make_async_copy(
                h_hbm.at[src_v.at[j]], rows_v.at[b], gsems.at[b]).wait()
            pltpu.sync_copy(rows_v.at[b], acc.at[dst_v.at[j]], add=True)

            @pl.when(g < n_outer - 1)
            def _():
                pltpu.async_copy(
                    h_hbm.at[src_v.at[j + NBUF]], rows_v.at[b], gsems.at[b])
        return carry

    lax.fori_loop(0, n_outer, outer, 0)
    plsc.subcore_barrier()

    pltpu.sync_copy(acc.at[pl.ds(r0, RPS)],
                    out_hbm.at[pl.ds(r0, RPS), pl.ds(c * HID, HID)])

    @pl.when(s == NS - 1)
    def _():
        pltpu.sync_copy(acc.at[pl.ds(TAIL0, TAILO)],
                        out_hbm.at[pl.ds(TAIL0, TAILO), pl.ds(c * HID, HID)])


@functools.cache
def _get_segsum():
    return pl.kernel(
        _segsum_body,
        out_type=jax.ShapeDtypeStruct((N_NODES, 2 * HID), _bf16),
        mesh=plsc.VectorSubcoreMesh(core_axis_name="c", subcore_axis_name="s"),
        scratch_types=[
            pltpu.VMEM((NCH, CH), jnp.int32),
            pltpu.VMEM((NCH, CH), jnp.int32),
            pltpu.VMEM((NBUF, CH, HID), _bf16),
            pltpu.VMEM_SHARED((ACC_ROWS, HID), _bf16),
            pltpu.SemaphoreType.DMA((NBUF,)),
        ],
        compiler_params=pltpu.CompilerParams(use_tc_tiling_on_sc=False),
    )


def _bn_relu(t, g, be):
    m = jnp.mean(t, axis=0, keepdims=True)
    v = jnp.mean(t * t, axis=0, keepdims=True) - m * m
    return jnp.maximum(g * (t - m) / jnp.sqrt(v + 1e-5) + be, 0.0)


def _mlp(y, w1, b1, g1, be1, w2, b2, g2, be2):
    t = jnp.dot(y, w1, preferred_element_type=_f32) + b1
    t = _bn_relu(t, g1, be1)
    u = jnp.dot(t, w2, preferred_element_type=_f32) + b2
    return _bn_relu(u, g2, be2)


def _onehot(batch):
    gids = lax.broadcasted_iota(jnp.int32, (N_NODES, N_GRAPHS), 1)
    return (batch == gids).astype(_f32)


def _pool_sum(h, wl, onehot):
    z = jnp.sum(h * wl, axis=1, keepdims=True)
    return lax.dot_general(z, onehot, (((0,), (0,)), ((), ())),
                           preferred_element_type=_f32)


def _first_body(x_ref, w1, b1, g1, be1, w2, b2, g2, be2, h_ref, hb_ref):
    h = _mlp(x_ref[...], w1[...], b1[...], g1[...], be1[...],
             w2[...], b2[...], g2[...], be2[...])
    h_ref[...] = h
    hb_ref[...] = h.astype(_bf16)


_first = pl.pallas_call(
    _first_body,
    out_shape=(jax.ShapeDtypeStruct((N_NODES, HID), _f32),
               jax.ShapeDtypeStruct((N_NODES, HID), _bf16)),
)


def _pool0_body(h_ref, wl, batch_ref, sc_ref):
    onehot = _onehot(batch_ref[...])
    z = jnp.sum(h_ref[...] * wl[...], axis=1, keepdims=True)
    zo = jnp.concatenate([z, jnp.ones_like(z)], axis=1)
    sc_ref[...] = lax.dot_general(zo, onehot, (((0,), (0,)), ((), ())),
                                  preferred_element_type=_f32)


_pool0 = pl.pallas_call(
    _pool0_body,
    out_shape=jax.ShapeDtypeStruct((2, N_GRAPHS), _f32),
)


def _pooln_body(h_ref, wl, batch_ref, s_ref):
    s_ref[...] = _pool_sum(h_ref[...], wl[...], _onehot(batch_ref[...]))


_pooln = pl.pallas_call(
    _pooln_body,
    out_shape=jax.ShapeDtypeStruct((1, N_GRAPHS), _f32),
)


def _mid_body(h_ref, p_ref, w1, b1, g1, be1, w2, b2, g2, be2,
              h_ref_out, hb_ref):
    p = p_ref[...].astype(_f32)
    y = h_ref[...] + p[:, :HID] + p[:, HID:]
    h = _mlp(y, w1[...], b1[...], g1[...], be1[...],
             w2[...], b2[...], g2[...], be2[...])
    h_ref_out[...] = h
    hb_ref[...] = h.astype(_bf16)


_mid = pl.pallas_call(
    _mid_body,
    out_shape=(jax.ShapeDtypeStruct((N_NODES, HID), _f32),
               jax.ShapeDtypeStruct((N_NODES, HID), _bf16)),
)


def _last_body(h_ref, p_ref, w1, b1, g1, be1, w2, b2, g2, be2, wl, batch_ref,
               sc_ref, s1_ref, s2_ref, bsum_ref, out_ref):
    p = p_ref[...].astype(_f32)
    y = h_ref[...] + p[:, :HID] + p[:, HID:]
    h = _mlp(y, w1[...], b1[...], g1[...], be1[...],
             w2[...], b2[...], g2[...], be2[...])
    s3 = _pool_sum(h, wl[...], _onehot(batch_ref[...]))
    s0 = sc_ref[0:1, :]
    cnt = sc_ref[1:2, :]
    total = s0 + s1_ref[...] + s2_ref[...] + s3 + bsum_ref[0, 0] * cnt
    out_ref[...] = jax.nn.sigmoid(total / jnp.maximum(cnt, 1.0))


_last = pl.pallas_call(
    _last_body,
    out_shape=jax.ShapeDtypeStruct((1, N_GRAPHS), _f32),
)


def kernel(x, edge_index, batch, params):
    npad = N_EDGES_PAD - N_EDGES
    spread = jnp.arange(npad, dtype=jnp.int32) % N_SCRAP
    srcp = jnp.concatenate(
        [edge_index[0], spread]).reshape(NW, NCH, CH)
    dstp = jnp.concatenate(
        [edge_index[1], N_NODES + spread]).reshape(NW, NCH, CH)
    zeros = jnp.zeros((ACC_ROWS, HID), _bf16)
    batch2 = batch.reshape(N_NODES, 1)
    bsum = (params['lin_b'][0] + params['lin_b'][1] + params['lin_b'][2]
            + params['lin_b'][3]).reshape(1, 1)

    def row(v):
        return v.reshape(1, HID)

    def mlp_args(p):
        return (p['W1'], row(p['b1']), row(p['g1']), row(p['be1']),
                p['W2'], row(p['b2']), row(p['g2']), row(p['be2']))

    segsum = _get_segsum()
    h, hb = _first(x, *mlp_args(params['first_h']))
    parts = segsum(hb, srcp, dstp, zeros)
    sc = _pool0(h, params['lin_W'][0].reshape(1, HID), batch2)
    h, hb = _mid(h, parts, *mlp_args(params['nns'][0]))
    parts = segsum(hb, srcp, dstp, zeros)
    s1 = _pooln(h, params['lin_W'][1].reshape(1, HID), batch2)
    h, hb = _mid(h, parts, *mlp_args(params['nns'][1]))
    parts = segsum(hb, srcp, dstp, zeros)
    s2 = _pooln(h, params['lin_W'][2].reshape(1, HID), batch2)
    out = _last(h, parts, *mlp_args(params['nns'][2]),
                params['lin_W'][3].reshape(1, HID), batch2,
                sc, s1, s2, bsum)
    return out.reshape(-1)
